# baseline (device time: 39057 ns/iter reference)
import jax
import jax.numpy as jnp
from jax import lax
from jax.experimental import pallas as pl
from jax.experimental.pallas import tpu as pltpu

N_ROWS = 2048
N_COLS = 1024
R = 128
NCHUNK = N_ROWS // R


def kernel(x, dest):
    z = (dest == 0)
    cz = jnp.cumsum(z.astype(jnp.int32))
    c0 = cz[-1]
    idx = jnp.arange(N_ROWS, dtype=jnp.int32)
    sorted_pos = jnp.where(z, cz - 1, c0 + idx - cz)
    q = jnp.where(sorted_pos >= c0, sorted_pos - c0, sorted_pos - c0 + N_ROWS)

    q2d = q.astype(jnp.int32).reshape(N_ROWS, 1)
    c0s = c0.astype(jnp.int32).reshape(1)

    def body(x_ref, q_ref, c0_ref, out_ref,
             bsend_ref, comm_ref, send_sems, recv_sems):
        my_y = lax.axis_index("y")
        peer = (lax.axis_index("x"), 1 - my_y)
        c0v = c0_ref[0]
        m = N_ROWS - c0v

        is0 = my_y == 0

        def send_cond(k):
            return (is0 & (k * R < m)) | (~is0 & (k >= m // R))

        def recv_cond(k):
            return (is0 & (k >= c0v // R)) | (~is0 & (k * R < c0v))

        def chunk_copy(k):
            return pltpu.make_async_remote_copy(
                src_ref=bsend_ref.at[pl.ds(k * R, R), :],
                dst_ref=comm_ref.at[pl.ds(k * R, R), :],
                send_sem=send_sems.at[k],
                recv_sem=recv_sems.at[k],
                device_id=peer,
                device_id_type=pl.DeviceIdType.MESH,
            )

        barrier = pltpu.get_barrier_semaphore()
        pl.semaphore_signal(
            barrier, inc=1, device_id=peer,
            device_id_type=pl.DeviceIdType.MESH,
        )
        pl.semaphore_wait(barrier, 1)

        xb = x_ref[...].astype(jnp.bfloat16)

        colio = lax.broadcasted_iota(jnp.int32, (N_ROWS, R), 1)
        qv = q_ref[...]
        for p in range(NCHUNK):
            kv = jnp.where(is0, p, NCHUNK - 1 - p).astype(jnp.int32)
            off = kv * R
            onehot = (qv == colio + off).astype(jnp.bfloat16)
            bsend_ref[pl.ds(off, R), :] = lax.dot_general(
                onehot, xb, (((0,), (0,)), ((), ())),
                preferred_element_type=jnp.float32,
            ).astype(jnp.bfloat16)

            @pl.when(send_cond(kv))
            def _(kv=kv):
                chunk_copy(kv).start()

        b = pltpu.roll(bsend_ref[...], c0v, axis=0)

        for k in range(NCHUNK):
            @pl.when(recv_cond(k))
            def _(k=k):
                chunk_copy(k).wait_recv()

        rowi = lax.broadcasted_iota(jnp.int32, (N_ROWS, 1), 0)
        keep = (rowi >= c0v) == (my_y == 1)
        out_ref[...] = jnp.where(keep, b, comm_ref[...]).astype(jnp.float32)

        for k in range(NCHUNK):
            @pl.when(send_cond(k))
            def _(k=k):
                chunk_copy(k).wait_send()

    return pl.pallas_call(
        body,
        out_shape=jax.ShapeDtypeStruct((N_ROWS, N_COLS), jnp.float32),
        in_specs=[
            pl.BlockSpec(memory_space=pltpu.VMEM),
            pl.BlockSpec(memory_space=pltpu.VMEM),
            pl.BlockSpec(memory_space=pltpu.SMEM),
        ],
        out_specs=pl.BlockSpec(memory_space=pltpu.VMEM),
        scratch_shapes=[
            pltpu.VMEM((N_ROWS, N_COLS), jnp.bfloat16),
            pltpu.VMEM((N_ROWS, N_COLS), jnp.bfloat16),
            pltpu.SemaphoreType.DMA((NCHUNK,)),
            pltpu.SemaphoreType.DMA((NCHUNK,)),
        ],
        compiler_params=pltpu.CompilerParams(collective_id=0),
    )(x, q2d, c0s)


# device time: 37733 ns/iter; 1.0351x vs baseline; 1.0351x over previous
import jax
import jax.numpy as jnp
from jax import lax
from jax.experimental import pallas as pl
from jax.experimental.pallas import tpu as pltpu

N_ROWS = 2048
N_COLS = 1024
R = 128
NCHUNK = N_ROWS // R


def kernel(x, dest):
    z = (dest == 0)
    cz = jnp.cumsum(z.astype(jnp.int32))
    c0 = cz[-1]
    idx = jnp.arange(N_ROWS, dtype=jnp.int32)
    sorted_pos = jnp.where(z, cz - 1, c0 + idx - cz)
    q = jnp.where(sorted_pos >= c0, sorted_pos - c0, sorted_pos - c0 + N_ROWS)

    q2d = q.astype(jnp.int32).reshape(1, N_ROWS)
    c0s = c0.astype(jnp.int32).reshape(1)

    def body(x_ref, q_ref, c0_ref, out_ref,
             bsend_ref, comm_ref, b_ref, send_sems, recv_sems):
        my_y = lax.axis_index("y")
        peer = (lax.axis_index("x"), 1 - my_y)
        c0v = c0_ref[0]
        m = N_ROWS - c0v

        is0 = my_y == 0

        def send_cond(k):
            return (is0 & (k * R < m)) | (~is0 & (k >= m // R))

        def recv_cond(k):
            return (is0 & (k >= c0v // R)) | (~is0 & (k * R < c0v))

        def chunk_copy(k):
            return pltpu.make_async_remote_copy(
                src_ref=bsend_ref.at[pl.ds(k * R, R), :],
                dst_ref=comm_ref.at[pl.ds(k * R, R), :],
                send_sem=send_sems.at[k],
                recv_sem=recv_sems.at[k],
                device_id=peer,
                device_id_type=pl.DeviceIdType.MESH,
            )

        barrier = pltpu.get_barrier_semaphore()
        pl.semaphore_signal(
            barrier, inc=1, device_id=peer,
            device_id_type=pl.DeviceIdType.MESH,
        )
        pl.semaphore_wait(barrier, 1)

        xb = x_ref[...].astype(jnp.bfloat16)

        rowio = lax.broadcasted_iota(jnp.int32, (R, N_ROWS), 0)
        qv = q_ref[...]
        for p in range(NCHUNK):
            kv = jnp.where(is0, p, NCHUNK - 1 - p).astype(jnp.int32)
            off = kv * R
            onehot = (qv == rowio + off).astype(jnp.bfloat16)
            bsend_ref[pl.ds(off, R), :] = jnp.dot(
                onehot, xb, preferred_element_type=jnp.float32
            ).astype(jnp.bfloat16)

            @pl.when(send_cond(kv))
            def _(kv=kv):
                chunk_copy(kv).start()

        b_ref[...] = pltpu.roll(bsend_ref[...], c0v, axis=0)

        crow = lax.broadcasted_iota(jnp.int32, (R, 1), 0)
        is1 = my_y == 1
        for p in range(NCHUNK):
            kvr = jnp.where(is0, NCHUNK - 1 - p, p).astype(jnp.int32)
            off = kvr * R

            @pl.when(recv_cond(kvr))
            def _(kvr=kvr):
                chunk_copy(kvr).wait_recv()

            keep = ((crow + off) >= c0v) == is1
            out_ref[pl.ds(off, R), :] = jnp.where(
                keep, b_ref[pl.ds(off, R), :], comm_ref[pl.ds(off, R), :]
            ).astype(jnp.float32)

        for k in range(NCHUNK):
            @pl.when(send_cond(k))
            def _(k=k):
                chunk_copy(k).wait_send()

    return pl.pallas_call(
        body,
        out_shape=jax.ShapeDtypeStruct((N_ROWS, N_COLS), jnp.float32),
        in_specs=[
            pl.BlockSpec(memory_space=pltpu.VMEM),
            pl.BlockSpec(memory_space=pltpu.VMEM),
            pl.BlockSpec(memory_space=pltpu.SMEM),
        ],
        out_specs=pl.BlockSpec(memory_space=pltpu.VMEM),
        scratch_shapes=[
            pltpu.VMEM((N_ROWS, N_COLS), jnp.bfloat16),
            pltpu.VMEM((N_ROWS, N_COLS), jnp.bfloat16),
            pltpu.VMEM((N_ROWS, N_COLS), jnp.bfloat16),
            pltpu.SemaphoreType.DMA((NCHUNK,)),
            pltpu.SemaphoreType.DMA((NCHUNK,)),
        ],
        compiler_params=pltpu.CompilerParams(collective_id=0),
    )(x, q2d, c0s)
